# Initial kernel scaffold; baseline (speedup 1.0000x reference)
import jax
import jax.numpy as jnp
from jax.experimental import pallas as pl


def kernel(input, embedding_weight):
    raise NotImplementedError("write your pallas kernel here")


# speedup vs baseline: 2.2905x; 2.2905x over previous
import jax
import jax.numpy as jnp
from jax.experimental import pallas as pl

_BLK = 512


def _bcast_kernel(w_ref, o_ref):
    o_ref[...] = jnp.broadcast_to(w_ref[...][None, :, :], o_ref.shape)


def kernel(input, embedding_weight):
    bsz, seq_len = input.shape
    d = embedding_weight.shape[1]
    nblk = seq_len // _BLK
    return pl.pallas_call(
        _bcast_kernel,
        grid=(nblk,),
        in_specs=[pl.BlockSpec((_BLK, d), lambda i: (i, 0))],
        out_specs=pl.BlockSpec((bsz, _BLK, d), lambda i: (0, i, 0)),
        out_shape=jax.ShapeDtypeStruct((bsz, seq_len, d), embedding_weight.dtype),
    )(embedding_weight[:seq_len])


# speedup vs baseline: 2.3570x; 1.0290x over previous
import jax
import jax.numpy as jnp
from jax.experimental import pallas as pl

_BLK = 1024


def _bcast_kernel(w_ref, o_ref):
    o_ref[...] = jnp.broadcast_to(w_ref[...][None, :, :], o_ref.shape)


def kernel(input, embedding_weight):
    bsz, seq_len = input.shape
    d = embedding_weight.shape[1]
    nblk = seq_len // _BLK
    return pl.pallas_call(
        _bcast_kernel,
        grid=(nblk,),
        in_specs=[pl.BlockSpec((_BLK, d), lambda i: (i, 0))],
        out_specs=pl.BlockSpec((bsz, _BLK, d), lambda i: (0, i, 0)),
        out_shape=jax.ShapeDtypeStruct((bsz, seq_len, d), embedding_weight.dtype),
    )(embedding_weight[:seq_len])
